# Initial kernel scaffold; baseline (speedup 1.0000x reference)
import jax
import jax.numpy as jnp
from jax.experimental import pallas as pl


def kernel(inputs, W_m, b_m, W_o, b_o):
    raise NotImplementedError("write your pallas kernel here")


# speedup vs baseline: 9.1804x; 9.1804x over previous
import functools

import jax
import jax.numpy as jnp
from jax.experimental import pallas as pl
from jax.experimental.pallas import tpu as pltpu

_C0 = 0.28209479177387814
_C1 = 0.4886025119029199
_K = 16


def _nbody_block(x_ref, w12_ref, wb_ref, bm_ref, woagg_ref, wox_ref, wv_ref,
                 bo_ref, out_ref, oh_ref, g_ref, *, bsz, n):
    f32 = jnp.float32
    blk = x_ref[...]
    mass = blk[:, :, 0:1]
    loc0 = blk[:, :, 1:4]
    vel = blk[:, :, 4:7]

    wsum = jnp.sum(mass, axis=1, keepdims=True)
    com = jnp.sum(mass / wsum * loc0, axis=1, keepdims=True)
    pos = loc0 - com

    diff = pos[:, :, None, :] - pos[:, None, :, :]
    d2 = jnp.sum(diff * diff, axis=-1)
    ii = jax.lax.broadcasted_iota(jnp.int32, (bsz, n, n), 1)
    jj = jax.lax.broadcasted_iota(jnp.int32, (bsz, n, n), 2)
    d2 = jnp.where(ii == jj, d2 + 1e10, d2)

    for k in range(_K):
        mn = jnp.min(d2, axis=-1, keepdims=True)
        cand = jnp.where(d2 <= mn, jj, jnp.int32(2**30))
        idx = jnp.min(cand, axis=-1, keepdims=True)
        eq = jj == idx
        oh_ref[:, k] = eq.astype(f32)
        d2 = jnp.where(eq, f32(1e30), d2)

    vabs = jnp.sqrt(jnp.sum(vel * vel, axis=-1, keepdims=True))
    feat = jnp.concatenate([pos, vel, vabs, mass], axis=-1)

    oh = oh_ref[...]
    for b in range(bsz):
        g_ref[b] = jnp.dot(oh[b].reshape(_K * n, n), feat[b],
                           preferred_element_type=f32)
    g = g_ref[...].reshape(bsz * _K * n, 8)

    pos_i = jnp.broadcast_to(pos[:, None], (bsz, _K, n, 3)).reshape(-1, 3)
    mass_i = jnp.broadcast_to(mass[:, None], (bsz, _K, n, 1)).reshape(-1, 1)
    pos_j = g[:, 0:3]
    mass_j = g[:, 7:8]
    rel = pos_j - pos_i
    dist = jnp.sqrt(jnp.sum(rel * rel, axis=-1, keepdims=True))
    u = rel / jnp.maximum(dist, 1e-8)
    mprod = mass_j * mass_i

    e12 = jnp.concatenate([g[:, 0:7], u, dist, mprod], axis=-1)
    x7 = feat[:, :, 0:7].reshape(bsz * n, 7)
    bx = jnp.dot(x7, wb_ref[...], preferred_element_type=f32)
    bx_t = jnp.broadcast_to(bx.reshape(bsz, 1, n, 128),
                            (bsz, _K, n, 128)).reshape(-1, 128)
    pre = jnp.dot(e12, w12_ref[...], preferred_element_type=f32) + bx_t
    m = jnp.maximum(pre + bm_ref[...], 0.0)

    agg = jnp.sum(m.reshape(bsz, _K, n, 128), axis=1).reshape(bsz * n, 128)
    usum = jnp.sum(u.reshape(bsz, _K, n, 3), axis=1).reshape(bsz * n, 3)

    uv = vel.reshape(bsz * n, 3) / jnp.maximum(vabs.reshape(bsz * n, 1), 1e-8)
    wvec = usum * (1.0 / _K) + uv

    shift = (jnp.dot(x7, wox_ref[...], preferred_element_type=f32)
             + jnp.dot(agg, woagg_ref[...], preferred_element_type=f32)
             + jnp.dot(wvec, wv_ref[...], preferred_element_type=f32)
             + bo_ref[...])

    com_t = jnp.broadcast_to(com, (bsz, n, 3)).reshape(bsz * n, 3)
    out = pos.reshape(bsz * n, 3) + shift + com_t
    out_ref[...] = out.reshape(bsz, n, 3)


def kernel(inputs, W_m, b_m, W_o, b_o):
    batchsize, n, _ = inputs.shape
    bsz = 4 if batchsize % 4 == 0 else 1
    f32 = jnp.float32

    w12 = jnp.concatenate(
        [W_m[0:7], _C1 * W_m[jnp.array([17, 15, 16])], W_m[18:20]], axis=0)
    wb = W_m[7:14]
    bm2 = (b_m + _C0 * W_m[14]).reshape(1, -1)
    wox = W_o[0:7]
    woagg = W_o[7:135]
    wv = _C1 * W_o[jnp.array([138, 136, 137])]
    bo2 = (b_o + 2.0 * _C0 * W_o[135]).reshape(1, -1)

    grid = (batchsize // bsz,)
    body = functools.partial(_nbody_block, bsz=bsz, n=n)
    preds = pl.pallas_call(
        body,
        grid=grid,
        in_specs=[
            pl.BlockSpec((bsz, n, 7), lambda i: (i, 0, 0)),
            pl.BlockSpec((12, 128), lambda i: (0, 0)),
            pl.BlockSpec((7, 128), lambda i: (0, 0)),
            pl.BlockSpec((1, 128), lambda i: (0, 0)),
            pl.BlockSpec((128, 3), lambda i: (0, 0)),
            pl.BlockSpec((7, 3), lambda i: (0, 0)),
            pl.BlockSpec((3, 3), lambda i: (0, 0)),
            pl.BlockSpec((1, 3), lambda i: (0, 0)),
        ],
        out_specs=pl.BlockSpec((bsz, n, 3), lambda i: (i, 0, 0)),
        out_shape=jax.ShapeDtypeStruct((batchsize, n, 3), f32),
        scratch_shapes=[
            pltpu.VMEM((bsz, _K, n, n), f32),
            pltpu.VMEM((bsz, _K * n, 8), f32),
        ],
        compiler_params=pltpu.CompilerParams(
            dimension_semantics=("arbitrary",)),
    )(inputs, w12, wb, bm2, woagg, wox, wv, bo2)
    return preds, jnp.zeros((batchsize,), dtype=f32)


# speedup vs baseline: 19.9870x; 2.1771x over previous
import functools

import jax
import jax.numpy as jnp
from jax.experimental import pallas as pl
from jax.experimental.pallas import tpu as pltpu

_C0 = 0.28209479177387814
_C1 = 0.4886025119029199
_K = 16


def _nbody_block(x_ref, w19_ref, bm_ref, woagg_ref, wox_ref, wv_ref,
                 bo_ref, out_ref, *, bsz, n):
    f32 = jnp.float32
    blk = x_ref[...]
    mass = blk[:, :, 0:1]
    loc = blk[:, :, 1:4]
    vel = blk[:, :, 4:7]

    wsum = jnp.sum(mass, axis=1, keepdims=True)
    com = jnp.sum(mass / wsum * loc, axis=1, keepdims=True)
    pos = loc - com
    vabs = jnp.sqrt(jnp.sum(vel * vel, axis=-1, keepdims=True))
    feat = jnp.concatenate([pos, vel, vabs, mass], axis=-1)

    jsub = jax.lax.broadcasted_iota(jnp.int32, (n, n), 0)
    ilane = jax.lax.broadcasted_iota(jnp.int32, (n, n), 1)
    diag = jsub == ilane

    for b in range(bsz):
        featT = jnp.transpose(feat[b], (1, 0))
        posT = featT[0:3]
        massT = featT[7:8]
        x7T = featT[0:7]

        d2T = None
        for c in range(3):
            pr = posT[c:c + 1, :]
            pc = jnp.transpose(pr, (1, 0))
            dc = pc - pr
            d2T = dc * dc if d2T is None else d2T + dc * dc
        d2T = jnp.where(diag, d2T + 1e10, d2T)

        accm = None
        accu = None
        for k in range(_K):
            mn = jnp.min(d2T, axis=0, keepdims=True)
            cand = jnp.where(d2T <= mn, jsub, jnp.int32(2**30))
            idx = jnp.min(cand, axis=0, keepdims=True)
            eqT = jsub == idx
            d2T = jnp.where(eqT, f32(1e30), d2T)

            g = jnp.dot(featT, eqT.astype(f32),
                        preferred_element_type=f32)
            relT = g[0:3] - posT
            d2e = jnp.sum(relT * relT, axis=0, keepdims=True)
            dist = jnp.sqrt(d2e)
            u = relT / jnp.maximum(dist, 1e-8)
            mprod = g[7:8] * massT
            e19 = jnp.concatenate([g[0:7], x7T, u, dist, mprod], axis=0)
            pre = jnp.dot(w19_ref[...], e19,
                          preferred_element_type=f32) + bm_ref[...]
            mk = jnp.maximum(pre, 0.0)
            accm = mk if accm is None else accm + mk
            accu = u if accu is None else accu + u

        uvT = featT[3:6] / jnp.maximum(featT[6:7], 1e-8)
        wvecT = accu * (1.0 / _K) + uvT
        shiftT = (jnp.dot(wox_ref[...], x7T, preferred_element_type=f32)
                  + jnp.dot(woagg_ref[...], accm, preferred_element_type=f32)
                  + jnp.dot(wv_ref[...], wvecT, preferred_element_type=f32)
                  + bo_ref[...])
        comT = jnp.transpose(com[b], (1, 0))
        outT = posT + shiftT + comT
        out_ref[b] = jnp.transpose(outT, (1, 0))


def kernel(inputs, W_m, b_m, W_o, b_o):
    batchsize, n, _ = inputs.shape
    bsz = 4 if batchsize % 4 == 0 else 1
    f32 = jnp.float32

    w19t = jnp.concatenate(
        [W_m[0:7], W_m[7:14], _C1 * W_m[jnp.array([17, 15, 16])],
         W_m[18:20]], axis=0).T
    bm2 = (b_m + _C0 * W_m[14]).reshape(-1, 1)
    woxt = W_o[0:7].T
    woaggt = W_o[7:135].T
    wvt = (_C1 * W_o[jnp.array([138, 136, 137])]).T
    bo2 = (b_o + 2.0 * _C0 * W_o[135]).reshape(-1, 1)

    grid = (batchsize // bsz,)
    body = functools.partial(_nbody_block, bsz=bsz, n=n)
    preds = pl.pallas_call(
        body,
        grid=grid,
        in_specs=[
            pl.BlockSpec((bsz, n, 7), lambda i: (i, 0, 0)),
            pl.BlockSpec((128, 19), lambda i: (0, 0)),
            pl.BlockSpec((128, 1), lambda i: (0, 0)),
            pl.BlockSpec((3, 128), lambda i: (0, 0)),
            pl.BlockSpec((3, 7), lambda i: (0, 0)),
            pl.BlockSpec((3, 3), lambda i: (0, 0)),
            pl.BlockSpec((3, 1), lambda i: (0, 0)),
        ],
        out_specs=pl.BlockSpec((bsz, n, 3), lambda i: (i, 0, 0)),
        out_shape=jax.ShapeDtypeStruct((batchsize, n, 3), f32),
        compiler_params=pltpu.CompilerParams(
            dimension_semantics=("arbitrary",)),
    )(inputs, w19t, bm2, woaggt, woxt, wvt, bo2)
    return preds, jnp.zeros((batchsize,), dtype=f32)


# speedup vs baseline: 24.4384x; 1.2227x over previous
import functools

import jax
import jax.numpy as jnp
from jax.experimental import pallas as pl
from jax.experimental.pallas import tpu as pltpu

_C0 = 0.28209479177387814
_C1 = 0.4886025119029199
_K = 16


def _nbody_block(x_ref, wsrc_ref, wdst_ref, w5_ref, bm_ref, woagg_ref,
                 wox_ref, wv_ref, bo_ref, out_ref, *, bsz, n):
    f32 = jnp.float32
    blk = x_ref[...]
    mass = blk[:, :, 0:1]
    loc = blk[:, :, 1:4]
    vel = blk[:, :, 4:7]

    wsum = jnp.sum(mass, axis=1, keepdims=True)
    com = jnp.sum(mass / wsum * loc, axis=1, keepdims=True)
    pos = loc - com
    vabs = jnp.sqrt(jnp.sum(vel * vel, axis=-1, keepdims=True))
    feat = jnp.concatenate([pos, vel, vabs, mass], axis=-1)

    jsub = jax.lax.broadcasted_iota(jnp.int32, (n, n), 0)
    diag = jsub == jax.lax.broadcasted_iota(jnp.int32, (n, n), 1)

    featT = [jnp.transpose(feat[b], (1, 0)) for b in range(bsz)]
    posT = [featT[b][0:3] for b in range(bsz)]
    massT = [featT[b][7:8] for b in range(bsz)]
    pmT = [jnp.concatenate([posT[b], massT[b]], axis=0) for b in range(bsz)]
    p_tab = [jnp.dot(wsrc_ref[...], featT[b][0:7],
                     preferred_element_type=f32) for b in range(bsz)]
    bxc = [jnp.dot(wdst_ref[...], featT[b][0:7],
                   preferred_element_type=f32) + bm_ref[...]
           for b in range(bsz)]

    d2T = []
    for b in range(bsz):
        acc = None
        for c in range(3):
            pr = posT[b][c:c + 1, :]
            pc = jnp.transpose(pr, (1, 0))
            dc = pc - pr
            acc = dc * dc if acc is None else acc + dc * dc
        d2T.append(jnp.where(diag, acc + 1e10, acc))

    accm = [None] * bsz
    accu = [None] * bsz
    for k in range(_K):
        for b in range(bsz):
            mn = jnp.min(d2T[b], axis=0, keepdims=True)
            cnd = jnp.where(d2T[b] <= mn, jsub, jnp.int32(2**30))
            idx = jnp.min(cnd, axis=0, keepdims=True)
            eqT = jsub == idx
            d2T[b] = jnp.where(eqT, f32(1e30), d2T[b])
            ohf = eqT.astype(f32)

            pk = jnp.dot(p_tab[b], ohf, preferred_element_type=f32)
            g4 = jnp.dot(pmT[b], ohf, preferred_element_type=f32)
            relT = g4[0:3] - posT[b]
            d2e = jnp.sum(relT * relT, axis=0, keepdims=True)
            dist = jnp.sqrt(d2e)
            u = relT * jax.lax.rsqrt(jnp.maximum(d2e, 1e-16))
            mprod = g4[3:4] * massT[b]
            udm = jnp.concatenate([u, dist, mprod], axis=0)
            qk = jnp.dot(w5_ref[...], udm, preferred_element_type=f32)
            mk = jnp.maximum(pk + qk + bxc[b], 0.0)
            accm[b] = mk if accm[b] is None else accm[b] + mk
            accu[b] = u if accu[b] is None else accu[b] + u

    for b in range(bsz):
        uvT = featT[b][3:6] / jnp.maximum(featT[b][6:7], 1e-8)
        wvecT = accu[b] * (1.0 / _K) + uvT
        shiftT = (jnp.dot(wox_ref[...], featT[b][0:7],
                          preferred_element_type=f32)
                  + jnp.dot(woagg_ref[...], accm[b],
                            preferred_element_type=f32)
                  + jnp.dot(wv_ref[...], wvecT, preferred_element_type=f32)
                  + bo_ref[...])
        comT = jnp.transpose(com[b], (1, 0))
        outT = posT[b] + shiftT + comT
        out_ref[b] = jnp.transpose(outT, (1, 0))


def kernel(inputs, W_m, b_m, W_o, b_o):
    batchsize, n, _ = inputs.shape
    bsz = 4 if batchsize % 4 == 0 else 1
    f32 = jnp.float32

    wsrct = W_m[0:7].T
    wdstt = W_m[7:14].T
    w5t = jnp.concatenate(
        [_C1 * W_m[jnp.array([17, 15, 16])], W_m[18:20]], axis=0).T
    bm2 = (b_m + _C0 * W_m[14]).reshape(-1, 1)
    woxt = W_o[0:7].T
    woaggt = W_o[7:135].T
    wvt = (_C1 * W_o[jnp.array([138, 136, 137])]).T
    bo2 = (b_o + 2.0 * _C0 * W_o[135]).reshape(-1, 1)

    grid = (batchsize // bsz,)
    body = functools.partial(_nbody_block, bsz=bsz, n=n)
    preds = pl.pallas_call(
        body,
        grid=grid,
        in_specs=[
            pl.BlockSpec((bsz, n, 7), lambda i: (i, 0, 0)),
            pl.BlockSpec((128, 7), lambda i: (0, 0)),
            pl.BlockSpec((128, 7), lambda i: (0, 0)),
            pl.BlockSpec((128, 5), lambda i: (0, 0)),
            pl.BlockSpec((128, 1), lambda i: (0, 0)),
            pl.BlockSpec((3, 128), lambda i: (0, 0)),
            pl.BlockSpec((3, 7), lambda i: (0, 0)),
            pl.BlockSpec((3, 3), lambda i: (0, 0)),
            pl.BlockSpec((3, 1), lambda i: (0, 0)),
        ],
        out_specs=pl.BlockSpec((bsz, n, 3), lambda i: (i, 0, 0)),
        out_shape=jax.ShapeDtypeStruct((batchsize, n, 3), f32),
        compiler_params=pltpu.CompilerParams(
            dimension_semantics=("arbitrary",)),
    )(inputs, wsrct, wdstt, w5t, bm2, woaggt, woxt, wvt, bo2)
    return preds, jnp.zeros((batchsize,), dtype=f32)
